# Initial kernel scaffold; baseline (speedup 1.0000x reference)
import jax
import jax.numpy as jnp
from jax.experimental import pallas as pl


def kernel(x, edge_index, W_l1, b_l1, W_r1, gamma, beta, W_l2, b_l2, W_r2):
    raise NotImplementedError("write your pallas kernel here")


# speedup vs baseline: 5.4432x; 5.4432x over previous
import functools

import jax
import jax.numpy as jnp
from jax import lax
from jax.experimental import pallas as pl
from jax.experimental.pallas import tpu as pltpu
from jax.experimental.pallas import tpu_sc as plsc

N_NODES = 10000
N_EDGES = 320000
D = 128
EPS = 1e-5

NC = 2
NS = 16
NW = NC * NS
K = 128
CHUNKS = 79
E_W = K * CHUNKS
E_PAD = E_W * NW
N_PAD = 10240
STRIPE = N_PAD // NS
PIECES = STRIPE // K


def _fill_rows(ref, vec16):
    def body(i, _):
        for j in range(D // 16):
            ref[i, pl.ds(j * 16, 16)] = vec16
        return 0
    lax.fori_loop(0, K, body, 0)


@functools.lru_cache(maxsize=None)
def _make_sc_agg():
    mesh = plsc.VectorSubcoreMesh(core_axis_name="c", subcore_axis_name="s",
                                  num_cores=NC, num_subcores=NS)

    def body(x_hbm, src_hbm, dst_hbm, part, idx_src, idx_dst, rows, acc_sh,
             sem):
        cid = lax.axis_index("c")
        sid = lax.axis_index("s")
        wid = cid * NS + sid
        base = sid * STRIPE

        _fill_rows(rows, jnp.zeros((16,), jnp.float32))
        for p in range(PIECES):
            pltpu.sync_copy(rows, acc_sh.at[pl.ds(base + p * K, K)])
        plsc.subcore_barrier()

        def chunk(c, _):
            row = wid * CHUNKS + c
            pltpu.sync_copy(src_hbm.at[row], idx_src)
            pltpu.sync_copy(dst_hbm.at[row], idx_dst)
            pltpu.async_copy(x_hbm.at[idx_src], rows, sem).wait()
            pltpu.sync_copy(rows, acc_sh.at[idx_dst], add=True)
            return 0
        lax.fori_loop(0, CHUNKS, chunk, 0)

        plsc.subcore_barrier()
        for p in range(PIECES):
            pltpu.sync_copy(acc_sh.at[pl.ds(base + p * K, K)], rows)
            pltpu.sync_copy(rows, part.at[cid, pl.ds(base + p * K, K)])

    return pl.kernel(
        body,
        out_type=jax.ShapeDtypeStruct((NC, N_PAD, D), jnp.float32),
        mesh=mesh,
        scratch_types=(
            pltpu.VMEM((K,), jnp.int32),
            pltpu.VMEM((K,), jnp.int32),
            pltpu.VMEM((K, D), jnp.float32),
            pltpu.VMEM_SHARED((N_PAD, D), jnp.float32),
            pltpu.SemaphoreType.DMA,
        ),
    )


@functools.lru_cache(maxsize=None)
def _make_sc_cnt():
    mesh = plsc.VectorSubcoreMesh(core_axis_name="c", subcore_axis_name="s",
                                  num_cores=NC, num_subcores=NS)

    def body(dst_hbm, cnt_out, idx_dst, ones, stage, cnt_sh, sem):
        cid = lax.axis_index("c")
        sid = lax.axis_index("s")
        wid = cid * NS + sid
        base = sid * STRIPE

        _fill_rows(stage, jnp.zeros((16,), jnp.float32))
        _fill_rows(ones, jnp.ones((16,), jnp.float32))
        for p in range(PIECES):
            pltpu.sync_copy(stage, cnt_sh.at[pl.ds(base + p * K, K)])
        plsc.subcore_barrier()

        def chunk(c, _):
            row = wid * CHUNKS + c
            pltpu.sync_copy(dst_hbm.at[row], idx_dst)
            pltpu.sync_copy(ones, cnt_sh.at[idx_dst], add=True)
            return 0
        lax.fori_loop(0, CHUNKS, chunk, 0)

        plsc.subcore_barrier()
        for p in range(PIECES):
            pltpu.sync_copy(cnt_sh.at[pl.ds(base + p * K, K)], stage)
            pltpu.sync_copy(stage, cnt_out.at[cid, pl.ds(base + p * K, K)])

    return pl.kernel(
        body,
        out_type=jax.ShapeDtypeStruct((NC, N_PAD, D), jnp.float32),
        mesh=mesh,
        scratch_types=(
            pltpu.VMEM((K,), jnp.int32),
            pltpu.VMEM((K, D), jnp.float32),
            pltpu.VMEM((K, D), jnp.float32),
            pltpu.VMEM_SHARED((N_PAD, D), jnp.float32),
            pltpu.SemaphoreType.DMA,
        ),
    )


def _tc_layer1(part, cnt, x, W_l1, b_l1, W_r1, gamma, beta):
    def body(part_ref, cnt_ref, x_ref, wl_ref, bl_ref, wr_ref, g_ref, b_ref,
             out_ref):
        agg = part_ref[0, :N_NODES, :] + part_ref[1, :N_NODES, :]
        c = cnt_ref[0, :N_NODES, 0:1] + cnt_ref[1, :N_NODES, 0:1]
        mean = agg * (1.0 / jnp.maximum(c, 1.0))
        dn = (((1,), (1,)), ((), ()))
        h = lax.dot_general(mean, wl_ref[...], dn,
                            preferred_element_type=jnp.float32)
        h = h + bl_ref[...][None, :]
        h = h + lax.dot_general(x_ref[...], wr_ref[...], dn,
                                preferred_element_type=jnp.float32)
        mu = jnp.mean(h, axis=0, keepdims=True)
        var = jnp.mean((h - mu) ** 2, axis=0, keepdims=True)
        hn = g_ref[...][None, :] * (h - mu) / jnp.sqrt(var + EPS)
        hn = hn + b_ref[...][None, :]
        out_ref[...] = jnp.maximum(hn, 0.0)

    return pl.pallas_call(
        body,
        out_shape=jax.ShapeDtypeStruct((N_NODES, D), jnp.float32),
    )(part, cnt, x, W_l1, b_l1, W_r1, gamma, beta)


def _tc_layer2(part, cnt, h, W_l2, b_l2, W_r2):
    def body(part_ref, cnt_ref, h_ref, wl_ref, bl_ref, wr_ref, out_ref):
        agg = part_ref[0, :N_NODES, :] + part_ref[1, :N_NODES, :]
        c = cnt_ref[0, :N_NODES, 0:1] + cnt_ref[1, :N_NODES, 0:1]
        mean = agg * (1.0 / jnp.maximum(c, 1.0))
        dn = (((1,), (1,)), ((), ()))
        out = lax.dot_general(mean, wl_ref[...], dn,
                              preferred_element_type=jnp.float32)
        out = out + bl_ref[...][None, :]
        out = out + lax.dot_general(h_ref[...], wr_ref[...], dn,
                                    preferred_element_type=jnp.float32)
        out_ref[...] = out

    return pl.pallas_call(
        body,
        out_shape=jax.ShapeDtypeStruct((N_NODES, D), jnp.float32),
    )(part, cnt, h, W_l2, b_l2, W_r2)


def kernel(x, edge_index, W_l1, b_l1, W_r1, gamma, beta, W_l2, b_l2, W_r2):
    src = edge_index[0].astype(jnp.int32)
    dst = edge_index[1].astype(jnp.int32)
    pad = E_PAD - N_EDGES
    fill = jnp.arange(pad, dtype=jnp.int32)
    src = jnp.concatenate([src, fill % N_NODES])
    dst = jnp.concatenate([dst, N_NODES + fill % (N_PAD - N_NODES)])
    src2 = src.reshape(NW * CHUNKS, K)
    dst2 = dst.reshape(NW * CHUNKS, K)

    cnt = _make_sc_cnt()(dst2)
    part1 = _make_sc_agg()(x, src2, dst2)
    h = _tc_layer1(part1, cnt, x, W_l1, b_l1, W_r1, gamma, beta)
    part2 = _make_sc_agg()(h, src2, dst2)
    out = _tc_layer2(part2, cnt, h, W_l2, b_l2, W_r2)
    return out


# speedup vs baseline: 7.7237x; 1.4189x over previous
import functools

import jax
import jax.numpy as jnp
from jax import lax
from jax.experimental import pallas as pl
from jax.experimental.pallas import tpu as pltpu
from jax.experimental.pallas import tpu_sc as plsc

N_NODES = 10000
N_EDGES = 320000
D = 128
EPS = 1e-5

NC = 2
NS = 16
NW = NC * NS
K = 128
CHUNKS = 80
PAIRS = CHUNKS // 2
E_W = K * CHUNKS
E_PAD = E_W * NW
N_PAD = 10240
STRIPE = N_PAD // NS
PIECES = STRIPE // K


def _fill_rows(ref, vec16):
    def body(i, _):
        for j in range(D // 16):
            ref[i, pl.ds(j * 16, 16)] = vec16
        return 0
    lax.fori_loop(0, K, body, 0)


@functools.lru_cache(maxsize=None)
def _make_sc_agg():
    mesh = plsc.VectorSubcoreMesh(core_axis_name="c", subcore_axis_name="s",
                                  num_cores=NC, num_subcores=NS)

    def body(x_hbm, src_hbm, dst_hbm, part, idx_src0, idx_src1, idx_dst0,
             idx_dst1, rows, acc_sh, sem0, sem1):
        cid = lax.axis_index("c")
        sid = lax.axis_index("s")
        wid = cid * NS + sid
        base = sid * STRIPE
        sems = (sem0, sem1)
        srcs = (idx_src0, idx_src1)
        dsts = (idx_dst0, idx_dst1)

        _fill_rows(rows.at[0], jnp.zeros((16,), jnp.float32))
        for p in range(PIECES):
            pltpu.sync_copy(rows.at[0], acc_sh.at[pl.ds(base + p * K, K)])
        plsc.subcore_barrier()

        def load_idx(c, b):
            row = wid * CHUNKS + c
            pltpu.sync_copy(src_hbm.at[row], srcs[b])
            pltpu.sync_copy(dst_hbm.at[row], dsts[b])

        def start_gather(b):
            pltpu.async_copy(x_hbm.at[srcs[b]], rows.at[b], sems[b])

        def finish_gather(b):
            pltpu.make_async_copy(x_hbm.at[srcs[b]], rows.at[b],
                                  sems[b]).wait()

        def scatter(b):
            pltpu.sync_copy(rows.at[b], acc_sh.at[dsts[b]], add=True)

        load_idx(0, 0)
        start_gather(0)

        def pair(i, _):
            c0 = 2 * i
            load_idx(c0 + 1, 1)
            start_gather(1)
            finish_gather(0)
            scatter(0)

            @pl.when(i < PAIRS - 1)
            def _():
                load_idx(c0 + 2, 0)
                start_gather(0)
            finish_gather(1)
            scatter(1)
            return 0
        lax.fori_loop(0, PAIRS, pair, 0)

        plsc.subcore_barrier()
        for p in range(PIECES):
            pltpu.sync_copy(acc_sh.at[pl.ds(base + p * K, K)], rows.at[0])
            pltpu.sync_copy(rows.at[0], part.at[cid, pl.ds(base + p * K, K)])

    return pl.kernel(
        body,
        out_type=jax.ShapeDtypeStruct((NC, N_PAD, D), jnp.float32),
        mesh=mesh,
        scratch_types=(
            pltpu.VMEM((K,), jnp.int32),
            pltpu.VMEM((K,), jnp.int32),
            pltpu.VMEM((K,), jnp.int32),
            pltpu.VMEM((K,), jnp.int32),
            pltpu.VMEM((2, K, D), jnp.float32),
            pltpu.VMEM_SHARED((N_PAD, D), jnp.float32),
            pltpu.SemaphoreType.DMA,
            pltpu.SemaphoreType.DMA,
        ),
    )


@functools.lru_cache(maxsize=None)
def _make_sc_cnt():
    mesh = plsc.VectorSubcoreMesh(core_axis_name="c", subcore_axis_name="s",
                                  num_cores=NC, num_subcores=NS)

    def body(dst_hbm, cnt_out, idx_dst, ones, stage, cnt_sh, sem):
        cid = lax.axis_index("c")
        sid = lax.axis_index("s")
        wid = cid * NS + sid
        base = sid * STRIPE

        _fill_rows(stage, jnp.zeros((16,), jnp.float32))
        _fill_rows(ones, jnp.ones((16,), jnp.float32))
        for p in range(PIECES):
            pltpu.sync_copy(stage, cnt_sh.at[pl.ds(base + p * K, K)])
        plsc.subcore_barrier()

        def chunk(c, _):
            row = wid * CHUNKS + c
            pltpu.sync_copy(dst_hbm.at[row], idx_dst)
            pltpu.sync_copy(ones, cnt_sh.at[idx_dst], add=True)
            return 0
        lax.fori_loop(0, CHUNKS, chunk, 0)

        plsc.subcore_barrier()
        for p in range(PIECES):
            pltpu.sync_copy(cnt_sh.at[pl.ds(base + p * K, K)], stage)
            pltpu.sync_copy(stage, cnt_out.at[cid, pl.ds(base + p * K, K)])

    return pl.kernel(
        body,
        out_type=jax.ShapeDtypeStruct((NC, N_PAD, D), jnp.float32),
        mesh=mesh,
        scratch_types=(
            pltpu.VMEM((K,), jnp.int32),
            pltpu.VMEM((K, D), jnp.float32),
            pltpu.VMEM((K, D), jnp.float32),
            pltpu.VMEM_SHARED((N_PAD, D), jnp.float32),
            pltpu.SemaphoreType.DMA,
        ),
    )


def _tc_layer1(part, cnt, x, W_l1, b_l1, W_r1, gamma, beta):
    def body(part_ref, cnt_ref, x_ref, wl_ref, bl_ref, wr_ref, g_ref, b_ref,
             out_ref):
        agg = part_ref[0, :N_NODES, :] + part_ref[1, :N_NODES, :]
        c = cnt_ref[0, :N_NODES, 0:1] + cnt_ref[1, :N_NODES, 0:1]
        mean = agg * (1.0 / jnp.maximum(c, 1.0))
        dn = (((1,), (1,)), ((), ()))
        h = lax.dot_general(mean, wl_ref[...], dn,
                            preferred_element_type=jnp.float32)
        h = h + bl_ref[...][None, :]
        h = h + lax.dot_general(x_ref[...], wr_ref[...], dn,
                                preferred_element_type=jnp.float32)
        mu = jnp.mean(h, axis=0, keepdims=True)
        var = jnp.mean((h - mu) ** 2, axis=0, keepdims=True)
        hn = g_ref[...][None, :] * (h - mu) / jnp.sqrt(var + EPS)
        hn = hn + b_ref[...][None, :]
        out_ref[...] = jnp.maximum(hn, 0.0)

    return pl.pallas_call(
        body,
        out_shape=jax.ShapeDtypeStruct((N_NODES, D), jnp.float32),
    )(part, cnt, x, W_l1, b_l1, W_r1, gamma, beta)


def _tc_layer2(part, cnt, h, W_l2, b_l2, W_r2):
    def body(part_ref, cnt_ref, h_ref, wl_ref, bl_ref, wr_ref, out_ref):
        agg = part_ref[0, :N_NODES, :] + part_ref[1, :N_NODES, :]
        c = cnt_ref[0, :N_NODES, 0:1] + cnt_ref[1, :N_NODES, 0:1]
        mean = agg * (1.0 / jnp.maximum(c, 1.0))
        dn = (((1,), (1,)), ((), ()))
        out = lax.dot_general(mean, wl_ref[...], dn,
                              preferred_element_type=jnp.float32)
        out = out + bl_ref[...][None, :]
        out = out + lax.dot_general(h_ref[...], wr_ref[...], dn,
                                    preferred_element_type=jnp.float32)
        out_ref[...] = out

    return pl.pallas_call(
        body,
        out_shape=jax.ShapeDtypeStruct((N_NODES, D), jnp.float32),
    )(part, cnt, h, W_l2, b_l2, W_r2)


def kernel(x, edge_index, W_l1, b_l1, W_r1, gamma, beta, W_l2, b_l2, W_r2):
    src = edge_index[0].astype(jnp.int32)
    dst = edge_index[1].astype(jnp.int32)
    pad = E_PAD - N_EDGES
    fill = jnp.arange(pad, dtype=jnp.int32)
    src = jnp.concatenate([src, fill % N_NODES])
    dst = jnp.concatenate([dst, N_NODES + fill % (N_PAD - N_NODES)])
    src2 = src.reshape(NW * CHUNKS, K)
    dst2 = dst.reshape(NW * CHUNKS, K)

    cnt = _make_sc_cnt()(dst2)
    part1 = _make_sc_agg()(x, src2, dst2)
    h = _tc_layer1(part1, cnt, x, W_l1, b_l1, W_r1, gamma, beta)
    part2 = _make_sc_agg()(h, src2, dst2)
    out = _tc_layer2(part2, cnt, h, W_l2, b_l2, W_r2)
    return out


# speedup vs baseline: 10.3494x; 1.3400x over previous
import functools

import jax
import jax.numpy as jnp
from jax import lax
from jax.experimental import pallas as pl
from jax.experimental.pallas import tpu as pltpu
from jax.experimental.pallas import tpu_sc as plsc

N_NODES = 10000
N_EDGES = 320000
D = 128
EPS = 1e-5

NC = 2
NS = 16
NW = NC * NS
K = 128
CHUNKS = 80
PAIRS = CHUNKS // 2
E_W = K * CHUNKS
E_PAD = E_W * NW
N_PAD = 10240
STRIPE = N_PAD // NS
PIECES = STRIPE // K


def _fill_rows(ref, vec16):
    def body(i, _):
        for j in range(D // 16):
            ref[i, pl.ds(j * 16, 16)] = vec16
        return 0
    lax.fori_loop(0, K, body, 0)


@functools.lru_cache(maxsize=None)
def _make_sc_agg():
    mesh = plsc.VectorSubcoreMesh(core_axis_name="c", subcore_axis_name="s",
                                  num_cores=NC, num_subcores=NS)

    def body(x_hbm, src_hbm, dst_hbm, part, idx_src0, idx_src1, idx_src2,
             idx_src3, idx_dst0, idx_dst1, idx_dst2, idx_dst3, rows, acc_sh,
             semg0, semg1, semi0, semi1, semi2, semi3):
        cid = lax.axis_index("c")
        sid = lax.axis_index("s")
        wid = cid * NS + sid
        base = sid * STRIPE
        semg = (semg0, semg1)
        semi = (semi0, semi1, semi2, semi3)
        srcs = (idx_src0, idx_src1, idx_src2, idx_src3)
        dsts = (idx_dst0, idx_dst1, idx_dst2, idx_dst3)

        _fill_rows(rows.at[0], jnp.zeros((16,), jnp.float32))
        for p in range(PIECES):
            pltpu.sync_copy(rows.at[0], acc_sh.at[pl.ds(base + p * K, K)])
        plsc.subcore_barrier()

        def load_idx_sync(c, s):
            row = wid * CHUNKS + c
            pltpu.sync_copy(src_hbm.at[row], srcs[s])
            pltpu.sync_copy(dst_hbm.at[row], dsts[s])

        def load_idx_async(c, s):
            row = wid * CHUNKS + c
            pltpu.async_copy(src_hbm.at[row], srcs[s], semi[s])
            pltpu.async_copy(dst_hbm.at[row], dsts[s], semi[s])

        def wait_idx(c, s):
            row = wid * CHUNKS + c
            pltpu.make_async_copy(src_hbm.at[row], srcs[s], semi[s]).wait()
            pltpu.make_async_copy(dst_hbm.at[row], dsts[s], semi[s]).wait()

        def start_gather(g, s):
            pltpu.async_copy(x_hbm.at[srcs[s]], rows.at[g], semg[g])

        def finish_gather(g, s):
            pltpu.make_async_copy(x_hbm.at[srcs[s]], rows.at[g],
                                  semg[g]).wait()

        def scatter(g, s):
            pltpu.sync_copy(rows.at[g], acc_sh.at[dsts[s]], add=True)

        load_idx_sync(0, 0)
        load_idx_sync(1, 1)
        start_gather(0, 0)
        if PAIRS > 1:
            load_idx_async(2, 2)
            load_idx_async(3, 3)

        def pair_body(i, a0, a1, b0, b1):
            c0 = 2 * i
            start_gather(1, a1)
            finish_gather(0, a0)
            scatter(0, a0)

            @pl.when(i < PAIRS - 1)
            def _():
                wait_idx(c0 + 2, b0)
                start_gather(0, b0)

            @pl.when(i < PAIRS - 2)
            def _():
                load_idx_async(c0 + 4, a0)
            finish_gather(1, a1)
            scatter(1, a1)

            @pl.when(i < PAIRS - 2)
            def _():
                load_idx_async(c0 + 5, a1)

            @pl.when(i < PAIRS - 1)
            def _():
                wait_idx(c0 + 3, b1)

        def two_pairs(j, _):
            pair_body(2 * j, 0, 1, 2, 3)
            pair_body(2 * j + 1, 2, 3, 0, 1)
            return 0
        lax.fori_loop(0, PAIRS // 2, two_pairs, 0)

        plsc.subcore_barrier()
        for p in range(PIECES):
            pltpu.sync_copy(acc_sh.at[pl.ds(base + p * K, K)], rows.at[0])
            pltpu.sync_copy(rows.at[0], part.at[cid, pl.ds(base + p * K, K)])

    return pl.kernel(
        body,
        out_type=jax.ShapeDtypeStruct((NC, N_PAD, D), jnp.float32),
        mesh=mesh,
        scratch_types=(
            pltpu.VMEM((K,), jnp.int32),
            pltpu.VMEM((K,), jnp.int32),
            pltpu.VMEM((K,), jnp.int32),
            pltpu.VMEM((K,), jnp.int32),
            pltpu.VMEM((K,), jnp.int32),
            pltpu.VMEM((K,), jnp.int32),
            pltpu.VMEM((K,), jnp.int32),
            pltpu.VMEM((K,), jnp.int32),
            pltpu.VMEM((2, K, D), jnp.float32),
            pltpu.VMEM_SHARED((N_PAD, D), jnp.float32),
            pltpu.SemaphoreType.DMA,
            pltpu.SemaphoreType.DMA,
            pltpu.SemaphoreType.DMA,
            pltpu.SemaphoreType.DMA,
            pltpu.SemaphoreType.DMA,
            pltpu.SemaphoreType.DMA,
        ),
    )


@functools.lru_cache(maxsize=None)
def _make_sc_cnt():
    mesh = plsc.VectorSubcoreMesh(core_axis_name="c", subcore_axis_name="s",
                                  num_cores=NC, num_subcores=NS)

    def body(dst_hbm, cnt_out, idx_dst0, idx_dst1, ones, stage, cnt_sh,
             semi0, semi1):
        cid = lax.axis_index("c")
        sid = lax.axis_index("s")
        wid = cid * NS + sid
        base = sid * STRIPE
        dsts = (idx_dst0, idx_dst1)
        semi = (semi0, semi1)

        _fill_rows(stage, jnp.zeros((16,), jnp.float32))
        _fill_rows(ones, jnp.ones((16,), jnp.float32))
        for p in range(PIECES):
            pltpu.sync_copy(stage, cnt_sh.at[pl.ds(base + p * K, K)])
        plsc.subcore_barrier()

        pltpu.sync_copy(dst_hbm.at[wid * CHUNKS], idx_dst0)
        if PAIRS > 0:
            pltpu.async_copy(dst_hbm.at[wid * CHUNKS + 1], idx_dst1, semi1)

        def half(i, c, s):
            row = wid * CHUNKS + c

            if s == 0:
                @pl.when(i > 0)
                def _():
                    pltpu.make_async_copy(dst_hbm.at[row], dsts[s],
                                          semi[s]).wait()
            else:
                pltpu.make_async_copy(dst_hbm.at[row], dsts[s],
                                      semi[s]).wait()
            pltpu.sync_copy(ones, cnt_sh.at[dsts[s]], add=True)

            @pl.when(c + 2 < CHUNKS)
            def _():
                pltpu.async_copy(dst_hbm.at[row + 2], dsts[s], semi[s])

        def pair(i, _):
            half(i, 2 * i, 0)
            half(i, 2 * i + 1, 1)
            return 0
        lax.fori_loop(0, PAIRS, pair, 0)

        plsc.subcore_barrier()
        for p in range(PIECES):
            pltpu.sync_copy(cnt_sh.at[pl.ds(base + p * K, K)], stage)
            pltpu.sync_copy(stage, cnt_out.at[cid, pl.ds(base + p * K, K)])

    return pl.kernel(
        body,
        out_type=jax.ShapeDtypeStruct((NC, N_PAD, D), jnp.float32),
        mesh=mesh,
        scratch_types=(
            pltpu.VMEM((K,), jnp.int32),
            pltpu.VMEM((K,), jnp.int32),
            pltpu.VMEM((K, D), jnp.float32),
            pltpu.VMEM((K, D), jnp.float32),
            pltpu.VMEM_SHARED((N_PAD, D), jnp.float32),
            pltpu.SemaphoreType.DMA,
            pltpu.SemaphoreType.DMA,
        ),
    )


def _tc_layer1(part, cnt, x, W_l1, b_l1, W_r1, gamma, beta):
    def body(part_ref, cnt_ref, x_ref, wl_ref, bl_ref, wr_ref, g_ref, b_ref,
             out_ref):
        agg = part_ref[0, :N_NODES, :] + part_ref[1, :N_NODES, :]
        c = cnt_ref[0, :N_NODES, 0:1] + cnt_ref[1, :N_NODES, 0:1]
        mean = agg * (1.0 / jnp.maximum(c, 1.0))
        dn = (((1,), (1,)), ((), ()))
        h = lax.dot_general(mean, wl_ref[...], dn,
                            preferred_element_type=jnp.float32)
        h = h + bl_ref[...][None, :]
        h = h + lax.dot_general(x_ref[...], wr_ref[...], dn,
                                preferred_element_type=jnp.float32)
        mu = jnp.mean(h, axis=0, keepdims=True)
        var = jnp.mean((h - mu) ** 2, axis=0, keepdims=True)
        hn = g_ref[...][None, :] * (h - mu) / jnp.sqrt(var + EPS)
        hn = hn + b_ref[...][None, :]
        out_ref[...] = jnp.maximum(hn, 0.0)

    return pl.pallas_call(
        body,
        out_shape=jax.ShapeDtypeStruct((N_NODES, D), jnp.float32),
    )(part, cnt, x, W_l1, b_l1, W_r1, gamma, beta)


def _tc_layer2(part, cnt, h, W_l2, b_l2, W_r2):
    def body(part_ref, cnt_ref, h_ref, wl_ref, bl_ref, wr_ref, out_ref):
        agg = part_ref[0, :N_NODES, :] + part_ref[1, :N_NODES, :]
        c = cnt_ref[0, :N_NODES, 0:1] + cnt_ref[1, :N_NODES, 0:1]
        mean = agg * (1.0 / jnp.maximum(c, 1.0))
        dn = (((1,), (1,)), ((), ()))
        out = lax.dot_general(mean, wl_ref[...], dn,
                              preferred_element_type=jnp.float32)
        out = out + bl_ref[...][None, :]
        out = out + lax.dot_general(h_ref[...], wr_ref[...], dn,
                                    preferred_element_type=jnp.float32)
        out_ref[...] = out

    return pl.pallas_call(
        body,
        out_shape=jax.ShapeDtypeStruct((N_NODES, D), jnp.float32),
    )(part, cnt, h, W_l2, b_l2, W_r2)


def kernel(x, edge_index, W_l1, b_l1, W_r1, gamma, beta, W_l2, b_l2, W_r2):
    src = edge_index[0].astype(jnp.int32)
    dst = edge_index[1].astype(jnp.int32)
    pad = E_PAD - N_EDGES
    fill = jnp.arange(pad, dtype=jnp.int32)
    src = jnp.concatenate([src, fill % N_NODES])
    dst = jnp.concatenate([dst, N_NODES + fill % (N_PAD - N_NODES)])
    src2 = src.reshape(NW * CHUNKS, K)
    dst2 = dst.reshape(NW * CHUNKS, K)

    cnt = _make_sc_cnt()(dst2)
    part1 = _make_sc_agg()(x, src2, dst2)
    h = _tc_layer1(part1, cnt, x, W_l1, b_l1, W_r1, gamma, beta)
    part2 = _make_sc_agg()(h, src2, dst2)
    out = _tc_layer2(part2, cnt, h, W_l2, b_l2, W_r2)
    return out
